# baseline (speedup 1.0000x reference)
import functools

import jax
import jax.numpy as jnp
from jax import lax
from jax.experimental import pallas as pl
from jax.experimental.pallas import tpu as pltpu
from jax.experimental.pallas import tpu_sc as plsc

NC = 2
NS = 16
NW = NC * NS


def _make_sc_kernel(E, W, B):
    WW = W // 4
    rows_pw = E // NW
    batch_pw = B // NW
    ids_pw = B // NS
    half = E // NC
    n_chunks = ids_pw // 16

    mesh = plsc.VectorSubcoreMesh(core_axis_name="c", subcore_axis_name="s")

    @functools.partial(
        pl.kernel,
        mesh=mesh,
        out_type=[
            jax.ShapeDtypeStruct((B, WW), jnp.int32),
            jax.ShapeDtypeStruct((E + 8, WW), jnp.int32),
        ],
        scratch_types=[
            pltpu.VMEM((batch_pw,), jnp.int32),
            pltpu.VMEM((batch_pw, WW), jnp.int32),
            pltpu.VMEM((ids_pw,), jnp.int32),
            pltpu.VMEM((2, 128), jnp.int32),
            pltpu.VMEM((ids_pw, WW), jnp.int32),
            pltpu.SemaphoreType.DMA,
        ],
    )
    def sc_k(focus_hbm, ids_hbm, ones_hbm, rowf_hbm, newf_hbm,
             idx_v, rows_v, ids2_v, scat_v, ones_v, sem):
        c = lax.axis_index("c")
        s = lax.axis_index("s")
        wid = s * NC + c

        b0 = wid * batch_pw
        pltpu.sync_copy(ids_hbm.at[pl.ds(b0, batch_pw)], idx_v)
        pltpu.async_copy(focus_hbm.at[idx_v], rows_v, sem).wait()
        pltpu.sync_copy(rows_v, rowf_hbm.at[pl.ds(b0, batch_pw)])

        e0 = c * half + s * rows_pw
        pltpu.sync_copy(focus_hbm.at[pl.ds(e0, rows_pw)], rows_v)
        pltpu.sync_copy(rows_v, newf_hbm.at[pl.ds(e0, rows_pw)])
        plsc.subcore_barrier()

        lo = c * half
        hi = lo + half
        pltpu.sync_copy(ids_hbm.at[pl.ds(s * ids_pw, ids_pw)], ids2_v)

        junk = jnp.full((16,), E, jnp.int32)
        for j in range(n_chunks):
            chunk = ids2_v[pl.ds(j * 16, 16)]
            ih = (chunk >= lo) & (chunk < hi)
            scat_v[j // 8, pl.ds((j % 8) * 16, 16)] = jnp.where(ih, chunk, junk)

        pltpu.sync_copy(ones_hbm.at[pl.ds(0, ids_pw)], ones_v)
        copy0 = pltpu.async_copy(
            ones_v.at[pl.ds(0, 128)], newf_hbm.at[scat_v.at[0]], sem)
        copy1 = pltpu.async_copy(
            ones_v.at[pl.ds(128, 128)], newf_hbm.at[scat_v.at[1]], sem)
        copy0.wait()
        copy1.wait()

    return sc_k


def _tc_body(obs_ref, rowf_ref, out_ref):
    x = obs_ref[...]
    r = x.shape[0]
    w = out_ref.shape[1]
    tl = lax.iota(jnp.int32, 128)[None, :]
    groups = []
    for o in range(w // 128):
        acc = jnp.zeros((r, 128), jnp.int32)
        for i in range(3):
            sl = x[:, 384 * o + 128 * i: 384 * o + 128 * (i + 1)]
            for j in range(3):
                pos = 3 * tl + j - 128 * i
                valid = (pos >= 0) & (pos < 128)
                idx = jnp.clip(pos, 0, 127)
                g = jnp.take_along_axis(
                    sl, jnp.broadcast_to(idx, (r, 128)), axis=1)
                acc = acc | jnp.where(
                    jnp.broadcast_to(valid, (r, 128)), g, jnp.int32(0))
        groups.append(acc)
    m = jnp.concatenate(groups, axis=1) & jnp.int32(0x7FFFFFFF)
    pad = m == 0
    rf = rowf_ref[...] != jnp.uint8(0)
    out_ref[...] = pad | (~rf)


def _tc_call(obs_i, row_focus, B, W):
    R = 256
    grid = (B // R,)
    return pl.pallas_call(
        _tc_body,
        grid=grid,
        in_specs=[
            pl.BlockSpec((R, 3 * W), lambda i: (i, 0)),
            pl.BlockSpec((R, W), lambda i: (i, 0)),
        ],
        out_specs=pl.BlockSpec((R, W), lambda i: (i, 0)),
        out_shape=jax.ShapeDtypeStruct((B, W), jnp.bool_),
    )(obs_i, row_focus)


def kernel(obs, focus_mask, training_env_ids):
    B, M, C = obs.shape
    E, W = focus_mask.shape

    WW = W // 4
    focus_i32 = lax.bitcast_convert_type(
        focus_mask.astype(jnp.uint8).reshape(E, WW, 4), jnp.int32)
    ones_i32 = jnp.full((B // NS, WW), 0x01010101, jnp.int32)

    sc_k = _make_sc_kernel(E, W, B)
    row_focus_i32, new_focus_i32 = sc_k(focus_i32, training_env_ids, ones_i32)

    row_focus_u8 = lax.bitcast_convert_type(
        row_focus_i32, jnp.uint8).reshape(B, W)

    obs_i = lax.bitcast_convert_type(obs, jnp.int32).reshape(B, M * C)
    obs_mask = _tc_call(obs_i, row_focus_u8, B, W)

    new_focus = lax.bitcast_convert_type(
        new_focus_i32[:E], jnp.uint8).reshape(E, W).astype(jnp.bool_)
    return obs, obs_mask, new_focus


# speedup vs baseline: 1.9807x; 1.9807x over previous
import functools

import jax
import jax.numpy as jnp
from jax import lax
from jax.experimental import pallas as pl
from jax.experimental.pallas import tpu as pltpu
from jax.experimental.pallas import tpu_sc as plsc

NC = 2
NS = 16
NW = NC * NS


def _make_sc_kernel(E, W, B):
    WW = W // 4
    rows_pw = E // NW
    batch_pw = B // NW
    ids_pw = B // NS
    half = E // NC
    n_chunks = ids_pw // 16

    mesh = plsc.VectorSubcoreMesh(core_axis_name="c", subcore_axis_name="s")

    @functools.partial(
        pl.kernel,
        mesh=mesh,
        out_type=[
            jax.ShapeDtypeStruct((B, WW), jnp.int32),
            jax.ShapeDtypeStruct((E + 2 * ids_pw, WW), jnp.int32),
        ],
        scratch_types=[
            pltpu.VMEM((batch_pw,), jnp.int32),
            pltpu.VMEM((batch_pw, WW), jnp.int32),
            pltpu.VMEM((rows_pw, WW), jnp.int32),
            pltpu.VMEM((ids_pw,), jnp.int32),
            pltpu.VMEM((2, 128), jnp.int32),
            pltpu.VMEM((ids_pw, WW), jnp.int32),
            pltpu.SemaphoreType.DMA,
            pltpu.SemaphoreType.DMA,
            pltpu.SemaphoreType.DMA,
        ],
    )
    def sc_k(focus_hbm, ids_hbm, ones_hbm, rowf_hbm, newf_hbm,
             idx_v, rows_v, cp_v, ids2_v, scat_v, ones_v, sem_g, sem_c, sem_o):
        c = lax.axis_index("c")
        s = lax.axis_index("s")
        wid = s * NC + c

        b0 = wid * batch_pw
        e0 = c * half + s * rows_pw
        pltpu.sync_copy(ids_hbm.at[pl.ds(b0, batch_pw)], idx_v)
        gather = pltpu.async_copy(focus_hbm.at[idx_v], rows_v, sem_g)
        cp_in = pltpu.async_copy(focus_hbm.at[pl.ds(e0, rows_pw)], cp_v, sem_c)
        ones_in = pltpu.async_copy(ones_hbm.at[pl.ds(0, ids_pw)], ones_v, sem_o)
        pltpu.sync_copy(ids_hbm.at[pl.ds(s * ids_pw, ids_pw)], ids2_v)

        lo = c * half
        hi = lo + half
        jbase = E + c * ids_pw
        for j in range(n_chunks):
            chunk = ids2_v[pl.ds(j * 16, 16)]
            ih = (chunk >= lo) & (chunk < hi)
            junk = jbase + 16 * j + lax.iota(jnp.int32, 16)
            scat_v[j // 8, pl.ds((j % 8) * 16, 16)] = jnp.where(ih, chunk, junk)

        cp_in.wait()
        pltpu.sync_copy(cp_v, newf_hbm.at[pl.ds(e0, rows_pw)])

        gather.wait()
        pltpu.sync_copy(rows_v, rowf_hbm.at[pl.ds(b0, batch_pw)])

        plsc.subcore_barrier()

        ones_in.wait()
        copy0 = pltpu.async_copy(
            ones_v.at[pl.ds(0, 128)], newf_hbm.at[scat_v.at[0]], sem_g)
        copy1 = pltpu.async_copy(
            ones_v.at[pl.ds(128, 128)], newf_hbm.at[scat_v.at[1]], sem_c)
        copy0.wait()
        copy1.wait()

    return sc_k


def _unpack_bytes(xw, w):
    r = xw.shape[0]
    groups = []
    for o in range(w // 128):
        lane = lax.iota(jnp.int32, 128)[None, :]
        widx = jnp.broadcast_to(32 * o + (lane // 4), (r, 128))
        g = jnp.take_along_axis(xw, widx, axis=1)
        sh = jnp.broadcast_to(8 * (lane % 4), (r, 128))
        groups.append((g >> sh) & jnp.int32(0xFF))
    return jnp.concatenate(groups, axis=1)


def _pack_body(focus_ref, out_ref):
    f = focus_ref[...].astype(jnp.int32)
    r, w = f.shape
    lane = lax.iota(jnp.int32, 128)[None, :]
    acc = jnp.zeros((r, 128), jnp.int32)
    for i in range(w // 128):
        sl = f[:, 128 * i: 128 * (i + 1)]
        for k in range(4):
            src = jnp.broadcast_to(4 * (lane - 32 * i) + k, (r, 128))
            g = jnp.take_along_axis(sl, jnp.clip(src, 0, 127), axis=1)
            valid = (lane >= 32 * i) & (lane < 32 * (i + 1))
            acc = acc | jnp.where(jnp.broadcast_to(valid, (r, 128)),
                                  g << (8 * k), jnp.int32(0))
    out_ref[...] = acc


def _tc_body(obs_ref, rowf_ref, newf_ref, mask_ref, nf_ref):
    x = lax.bitcast_convert_type(obs_ref[...], jnp.int32)
    r = x.shape[0]
    w = mask_ref.shape[1]
    tl = lax.iota(jnp.int32, 128)[None, :]
    groups = []
    for o in range(w // 128):
        acc = jnp.zeros((r, 128), jnp.int32)
        for i in range(3):
            sl = x[:, 384 * o + 128 * i: 384 * o + 128 * (i + 1)]
            for j in range(3):
                pos = 3 * tl + j - 128 * i
                valid = (pos >= 0) & (pos < 128)
                idx = jnp.clip(pos, 0, 127)
                g = jnp.take_along_axis(
                    sl, jnp.broadcast_to(idx, (r, 128)), axis=1)
                acc = acc | jnp.where(
                    jnp.broadcast_to(valid, (r, 128)), g, jnp.int32(0))
        groups.append(acc)
    m = jnp.concatenate(groups, axis=1) & jnp.int32(0x7FFFFFFF)
    pad = m == 0
    rf = _unpack_bytes(rowf_ref[...], w)
    mask_ref[...] = pad | (rf == 0)
    nf_ref[...] = _unpack_bytes(newf_ref[...], w) != 0


def kernel(obs, focus_mask, training_env_ids):
    B, M, C = obs.shape
    E, W = focus_mask.shape
    WW = W // 4
    R = 256

    focus_i32 = lax.bitcast_convert_type(
        focus_mask.astype(jnp.uint8).reshape(E, WW, 4), jnp.int32)

    ones_i32 = jnp.full((B // NS, WW), 0x01010101, jnp.int32)
    sc_k = _make_sc_kernel(E, W, B)
    row_focus_i32, new_focus_i32 = sc_k(focus_i32, training_env_ids, ones_i32)

    obs_i = obs.reshape(B, M * C)
    obs_mask, new_focus = pl.pallas_call(
        _tc_body,
        grid=(B // R,),
        in_specs=[
            pl.BlockSpec((R, 3 * W), lambda i: (i, 0)),
            pl.BlockSpec((R, WW), lambda i: (i, 0)),
            pl.BlockSpec((R, WW), lambda i: (i, 0)),
        ],
        out_specs=[
            pl.BlockSpec((R, W), lambda i: (i, 0)),
            pl.BlockSpec((R, W), lambda i: (i, 0)),
        ],
        out_shape=[
            jax.ShapeDtypeStruct((B, W), jnp.bool_),
            jax.ShapeDtypeStruct((E, W), jnp.bool_),
        ],
    )(obs_i, row_focus_i32, new_focus_i32)

    return obs, obs_mask, new_focus


# speedup vs baseline: 2.0228x; 1.0213x over previous
import functools

import jax
import jax.numpy as jnp
from jax import lax
from jax.experimental import pallas as pl
from jax.experimental.pallas import tpu as pltpu
from jax.experimental.pallas import tpu_sc as plsc

NC = 2
NS = 16
NW = NC * NS


def _make_sc_kernel(E, W, B):
    WW = W // 4
    rows_pw = E // NW
    batch_pw = B // NW
    ids_pw = B // NS
    half = E // NC
    n_chunks = ids_pw // 16

    mesh = plsc.VectorSubcoreMesh(core_axis_name="c", subcore_axis_name="s")

    @functools.partial(
        pl.kernel,
        mesh=mesh,
        out_type=[
            jax.ShapeDtypeStruct((B, WW), jnp.int32),
            jax.ShapeDtypeStruct((E + 2 * ids_pw, WW), jnp.int32),
        ],
        scratch_types=[
            pltpu.VMEM((batch_pw,), jnp.int32),
            pltpu.VMEM((batch_pw, WW), jnp.int32),
            pltpu.VMEM((rows_pw, WW), jnp.int32),
            pltpu.VMEM((ids_pw,), jnp.int32),
            pltpu.VMEM((2, 128), jnp.int32),
            pltpu.VMEM((ids_pw, WW), jnp.int32),
            pltpu.SemaphoreType.DMA,
            pltpu.SemaphoreType.DMA,
            pltpu.SemaphoreType.DMA,
        ],
    )
    def sc_k(focus_hbm, ids_hbm, ones_hbm, rowf_hbm, newf_hbm,
             idx_v, rows_v, cp_v, ids2_v, scat_v, ones_v, sem_g, sem_c, sem_o):
        c = lax.axis_index("c")
        s = lax.axis_index("s")
        wid = s * NC + c

        b0 = wid * batch_pw
        e0 = c * half + s * rows_pw
        pltpu.sync_copy(ids_hbm.at[pl.ds(b0, batch_pw)], idx_v)
        gather = pltpu.async_copy(focus_hbm.at[idx_v], rows_v, sem_g)
        cp_in = pltpu.async_copy(focus_hbm.at[pl.ds(e0, rows_pw)], cp_v, sem_c)
        ones_in = pltpu.async_copy(ones_hbm.at[pl.ds(0, ids_pw)], ones_v, sem_o)
        pltpu.sync_copy(ids_hbm.at[pl.ds(s * ids_pw, ids_pw)], ids2_v)

        lo = c * half
        hi = lo + half
        jbase = E + c * ids_pw
        for j in range(n_chunks):
            chunk = ids2_v[pl.ds(j * 16, 16)]
            ih = (chunk >= lo) & (chunk < hi)
            junk = jbase + 16 * j + lax.iota(jnp.int32, 16)
            scat_v[j // 8, pl.ds((j % 8) * 16, 16)] = jnp.where(ih, chunk, junk)

        cp_in.wait()
        pltpu.sync_copy(cp_v, newf_hbm.at[pl.ds(e0, rows_pw)])

        gather.wait()
        pltpu.sync_copy(rows_v, rowf_hbm.at[pl.ds(b0, batch_pw)])

        plsc.subcore_barrier()

        ones_in.wait()
        copy0 = pltpu.async_copy(
            ones_v.at[pl.ds(0, 128)], newf_hbm.at[scat_v.at[0]], sem_g)
        copy1 = pltpu.async_copy(
            ones_v.at[pl.ds(128, 128)], newf_hbm.at[scat_v.at[1]], sem_c)
        copy0.wait()
        copy1.wait()

    return sc_k


def _unpack_bytes(xw, w):
    r = xw.shape[0]
    groups = []
    for o in range(w // 128):
        lane = lax.iota(jnp.int32, 128)[None, :]
        widx = jnp.broadcast_to(32 * o + (lane // 4), (r, 128))
        g = jnp.take_along_axis(xw, widx, axis=1)
        sh = jnp.broadcast_to(8 * (lane % 4), (r, 128))
        groups.append((g >> sh) & jnp.int32(0xFF))
    return jnp.concatenate(groups, axis=1)


def _pack_body(focus_ref, out_ref):
    f = focus_ref[...].astype(jnp.int32)
    r, w = f.shape
    lane = lax.iota(jnp.int32, 128)[None, :]
    acc = jnp.zeros((r, 128), jnp.int32)
    for i in range(w // 128):
        sl = f[:, 128 * i: 128 * (i + 1)]
        for k in range(4):
            src = jnp.broadcast_to(4 * (lane - 32 * i) + k, (r, 128))
            g = jnp.take_along_axis(sl, jnp.clip(src, 0, 127), axis=1)
            valid = (lane >= 32 * i) & (lane < 32 * (i + 1))
            acc = acc | jnp.where(jnp.broadcast_to(valid, (r, 128)),
                                  g << (8 * k), jnp.int32(0))
    out_ref[...] = acc


def _tc_body(obs_ref, rowf_ref, newf_ref, mask_ref, nf_ref):
    x = lax.bitcast_convert_type(obs_ref[...], jnp.int32)
    r = x.shape[0]
    w = mask_ref.shape[1]
    tl = lax.iota(jnp.int32, 128)[None, :]
    groups = []
    for o in range(w // 128):
        acc = jnp.zeros((r, 128), jnp.int32)
        for i in range(3):
            sl = x[:, 384 * o + 128 * i: 384 * o + 128 * (i + 1)]
            for j in range(3):
                pos = 3 * tl + j - 128 * i
                valid = (pos >= 0) & (pos < 128)
                idx = jnp.clip(pos, 0, 127)
                g = jnp.take_along_axis(
                    sl, jnp.broadcast_to(idx, (r, 128)), axis=1)
                acc = acc | jnp.where(
                    jnp.broadcast_to(valid, (r, 128)), g, jnp.int32(0))
        groups.append(acc)
    m = jnp.concatenate(groups, axis=1) & jnp.int32(0x7FFFFFFF)
    pad = m == 0
    rf = _unpack_bytes(rowf_ref[...], w)
    mask_ref[...] = (pad | (rf == 0)).astype(jnp.uint8)
    nf_ref[...] = (_unpack_bytes(newf_ref[...], w) != 0).astype(jnp.uint8)


def kernel(obs, focus_mask, training_env_ids):
    B, M, C = obs.shape
    E, W = focus_mask.shape
    WW = W // 4
    R = 256

    focus_i32 = lax.bitcast_convert_type(
        focus_mask.astype(jnp.uint8).reshape(E, WW, 4), jnp.int32)

    ones_i32 = jnp.full((B // NS, WW), 0x01010101, jnp.int32)
    sc_k = _make_sc_kernel(E, W, B)
    row_focus_i32, new_focus_i32 = sc_k(focus_i32, training_env_ids, ones_i32)

    obs_i = obs.reshape(B, M * C)
    obs_mask, new_focus = pl.pallas_call(
        _tc_body,
        grid=(B // R,),
        in_specs=[
            pl.BlockSpec((R, 3 * W), lambda i: (i, 0)),
            pl.BlockSpec((R, WW), lambda i: (i, 0)),
            pl.BlockSpec((R, WW), lambda i: (i, 0)),
        ],
        out_specs=[
            pl.BlockSpec((R, W), lambda i: (i, 0)),
            pl.BlockSpec((R, W), lambda i: (i, 0)),
        ],
        out_shape=[
            jax.ShapeDtypeStruct((B, W), jnp.uint8),
            jax.ShapeDtypeStruct((E, W), jnp.uint8),
        ],
    )(obs_i, row_focus_i32, new_focus_i32)

    return obs, obs_mask.astype(jnp.bool_), new_focus.astype(jnp.bool_)


# speedup vs baseline: 2.0329x; 1.0050x over previous
import functools

import jax
import jax.numpy as jnp
from jax import lax
from jax.experimental import pallas as pl
from jax.experimental.pallas import tpu as pltpu
from jax.experimental.pallas import tpu_sc as plsc

NC = 2
NS = 16
NW = NC * NS


def _make_sc_kernel(E, W, B):
    WW = W // 4
    rows_pw = E // NW
    batch_pw = B // NW
    ids_pw = B // NS
    half = E // NC
    n_chunks = ids_pw // 16

    mesh = plsc.VectorSubcoreMesh(core_axis_name="c", subcore_axis_name="s")

    @functools.partial(
        pl.kernel,
        mesh=mesh,
        out_type=[
            jax.ShapeDtypeStruct((B, WW), jnp.int32),
            jax.ShapeDtypeStruct((E + 2 * ids_pw, WW), jnp.int32),
        ],
        scratch_types=[
            pltpu.VMEM((batch_pw,), jnp.int32),
            pltpu.VMEM((batch_pw, WW), jnp.int32),
            pltpu.VMEM((rows_pw, WW), jnp.int32),
            pltpu.VMEM((ids_pw,), jnp.int32),
            pltpu.VMEM((2, 128), jnp.int32),
            pltpu.VMEM((ids_pw, WW), jnp.int32),
            pltpu.SemaphoreType.DMA,
            pltpu.SemaphoreType.DMA,
            pltpu.SemaphoreType.DMA,
        ],
    )
    def sc_k(focus_hbm, ids_hbm, ones_hbm, rowf_hbm, newf_hbm,
             idx_v, rows_v, cp_v, ids2_v, scat_v, ones_v, sem_g, sem_c, sem_o):
        c = lax.axis_index("c")
        s = lax.axis_index("s")
        wid = s * NC + c

        b0 = wid * batch_pw
        e0 = c * half + s * rows_pw
        pltpu.sync_copy(ids_hbm.at[pl.ds(b0, batch_pw)], idx_v)
        gather = pltpu.async_copy(focus_hbm.at[idx_v], rows_v, sem_g)
        cp_in = pltpu.async_copy(focus_hbm.at[pl.ds(e0, rows_pw)], cp_v, sem_c)
        ones_in = pltpu.async_copy(ones_hbm.at[pl.ds(0, ids_pw)], ones_v, sem_o)
        pltpu.sync_copy(ids_hbm.at[pl.ds(s * ids_pw, ids_pw)], ids2_v)

        lo = c * half
        hi = lo + half
        jbase = E + c * ids_pw
        for j in range(n_chunks):
            chunk = ids2_v[pl.ds(j * 16, 16)]
            ih = (chunk >= lo) & (chunk < hi)
            junk = jbase + 16 * j + lax.iota(jnp.int32, 16)
            scat_v[j // 8, pl.ds((j % 8) * 16, 16)] = jnp.where(ih, chunk, junk)

        cp_in.wait()
        pltpu.sync_copy(cp_v, newf_hbm.at[pl.ds(e0, rows_pw)])

        gather.wait()
        pltpu.sync_copy(rows_v, rowf_hbm.at[pl.ds(b0, batch_pw)])

        plsc.subcore_barrier()

        ones_in.wait()
        copy0 = pltpu.async_copy(
            ones_v.at[pl.ds(0, 128)], newf_hbm.at[scat_v.at[0]], sem_g)
        copy1 = pltpu.async_copy(
            ones_v.at[pl.ds(128, 128)], newf_hbm.at[scat_v.at[1]], sem_c)
        copy0.wait()
        copy1.wait()

    return sc_k


def _unpack_bytes(xw, w):
    r = xw.shape[0]
    groups = []
    for o in range(w // 128):
        lane = lax.iota(jnp.int32, 128)[None, :]
        widx = jnp.broadcast_to(32 * o + (lane // 4), (r, 128))
        g = jnp.take_along_axis(xw, widx, axis=1)
        sh = jnp.broadcast_to(8 * (lane % 4), (r, 128))
        groups.append((g >> sh) & jnp.int32(0xFF))
    return jnp.concatenate(groups, axis=1)


def _pack_body(focus_ref, out_ref):
    f = focus_ref[...].astype(jnp.int32)
    r, w = f.shape
    lane = lax.iota(jnp.int32, 128)[None, :]
    acc = jnp.zeros((r, 128), jnp.int32)
    for i in range(w // 128):
        sl = f[:, 128 * i: 128 * (i + 1)]
        for k in range(4):
            src = jnp.broadcast_to(4 * (lane - 32 * i) + k, (r, 128))
            g = jnp.take_along_axis(sl, jnp.clip(src, 0, 127), axis=1)
            valid = (lane >= 32 * i) & (lane < 32 * (i + 1))
            acc = acc | jnp.where(jnp.broadcast_to(valid, (r, 128)),
                                  g << (8 * k), jnp.int32(0))
    out_ref[...] = acc


def _tc_body(obs_ref, rowf_ref, newf_ref, mask_ref, nf_ref):
    x = lax.bitcast_convert_type(obs_ref[...], jnp.int32)
    r = x.shape[0]
    w = mask_ref.shape[1]
    tl = lax.iota(jnp.int32, 128)[None, :]
    groups = []
    for o in range(w // 128):
        acc = jnp.zeros((r, 128), jnp.int32)
        for i in range(3):
            sl = x[:, 384 * o + 128 * i: 384 * o + 128 * (i + 1)]
            for j in range(3):
                pos = 3 * tl + j - 128 * i
                valid = (pos >= 0) & (pos < 128)
                idx = jnp.clip(pos, 0, 127)
                g = jnp.take_along_axis(
                    sl, jnp.broadcast_to(idx, (r, 128)), axis=1)
                acc = acc | jnp.where(
                    jnp.broadcast_to(valid, (r, 128)), g, jnp.int32(0))
        groups.append(acc)
    m = jnp.concatenate(groups, axis=1) & jnp.int32(0x7FFFFFFF)
    pad = m == 0
    rf = _unpack_bytes(rowf_ref[...], w)
    mask_ref[...] = (pad | (rf == 0)).astype(jnp.uint8)
    nf_ref[...] = (_unpack_bytes(newf_ref[...], w) != 0).astype(jnp.uint8)


def kernel(obs, focus_mask, training_env_ids):
    B, M, C = obs.shape
    E, W = focus_mask.shape
    WW = W // 4
    R = 512

    focus_i32 = lax.bitcast_convert_type(
        focus_mask.astype(jnp.uint8).reshape(E, WW, 4), jnp.int32)

    ones_i32 = jnp.full((B // NS, WW), 0x01010101, jnp.int32)
    sc_k = _make_sc_kernel(E, W, B)
    row_focus_i32, new_focus_i32 = sc_k(focus_i32, training_env_ids, ones_i32)

    obs_i = obs.reshape(B, M * C)
    obs_mask, new_focus = pl.pallas_call(
        _tc_body,
        grid=(B // R,),
        in_specs=[
            pl.BlockSpec((R, 3 * W), lambda i: (i, 0)),
            pl.BlockSpec((R, WW), lambda i: (i, 0)),
            pl.BlockSpec((R, WW), lambda i: (i, 0)),
        ],
        out_specs=[
            pl.BlockSpec((R, W), lambda i: (i, 0)),
            pl.BlockSpec((R, W), lambda i: (i, 0)),
        ],
        out_shape=[
            jax.ShapeDtypeStruct((B, W), jnp.uint8),
            jax.ShapeDtypeStruct((E, W), jnp.uint8),
        ],
    )(obs_i, row_focus_i32, new_focus_i32)

    return obs, obs_mask.astype(jnp.bool_), new_focus.astype(jnp.bool_)
